# baseline (speedup 1.0000x reference)
import functools

import jax
import jax.numpy as jnp
from jax import lax
from jax.experimental import pallas as pl
from jax.experimental.pallas import tpu as pltpu
from jax.experimental.pallas import tpu_sc as plsc

N = 32768
P = 2048
B = 16
K = 20
DIN = 3
DH = 64

R = 1024
EPS = 1e-5


def _knn_body(xq_ref, xat_ref, tW_ref, tb_ref, idx_ref, h_ref, st_ref):
    b = pl.program_id(0)
    xq = xq_ref[0]
    xat = xat_ref[0]

    h = jnp.dot(xq, tW_ref[...], preferred_element_type=jnp.float32)
    h = jnp.maximum(h + tb_ref[...], 0.0)
    h_ref[0, :, 0:DH] = h

    @pl.when(jnp.logical_and(b == 0, pl.program_id(1) == 0))
    def _():
        st_ref[...] = jnp.zeros_like(st_ref)

    st_ref[0:1, :] += jnp.sum(h, axis=0, keepdims=True)
    st_ref[1:2, :] += jnp.sum(h * h, axis=0, keepdims=True)
    d2 = ((xq[:, 0:1] - xat[0:1, :]) ** 2
          + (xq[:, 1:2] - xat[1:2, :]) ** 2
          + (xq[:, 2:3] - xat[2:3, :]) ** 2)
    colio = lax.broadcasted_iota(jnp.int32, (R, P), 1)
    colf = colio.astype(jnp.float32)
    r = pl.program_id(1)
    selfcol = r * R + lax.broadcasted_iota(jnp.int32, (R, 1), 0)
    d2 = jnp.where(colio == selfcol, jnp.inf, d2)
    cols = [selfcol.astype(jnp.float32)]
    for k in range(K - 1):
        m = jnp.min(d2, axis=1, keepdims=True)
        eq = d2 == m
        t = jnp.where(eq, colf, float(P))
        a = jnp.min(t, axis=1, keepdims=True)
        cols.append(a)
        if k < K - 2:
            d2 = jnp.where(eq, jnp.inf, d2)
    idx = jnp.concatenate(cols, axis=1).astype(jnp.int32)
    idx_ref[0] = idx + b * P


def _knn(x, tW, tb):
    xr = x.reshape(B, P, DIN)
    xat = xr.transpose(0, 2, 1)
    idx, h, st = pl.pallas_call(
        _knn_body,
        grid=(B, P // R),
        in_specs=[
            pl.BlockSpec((1, R, DIN), lambda b, r: (b, r, 0)),
            pl.BlockSpec((1, DIN, P), lambda b, r: (b, 0, 0)),
            pl.BlockSpec((DIN, DH), lambda b, r: (0, 0)),
            pl.BlockSpec((1, DH), lambda b, r: (0, 0)),
        ],
        out_specs=[
            pl.BlockSpec((1, R, K), lambda b, r: (b, r, 0)),
            pl.BlockSpec((1, R, 2 * DH), lambda b, r: (b, r, 0)),
            pl.BlockSpec((8, DH), lambda b, r: (0, 0)),
        ],
        out_shape=[
            jax.ShapeDtypeStruct((B, P, K), jnp.int32),
            jax.ShapeDtypeStruct((B, P, 2 * DH), jnp.float32),
            jax.ShapeDtypeStruct((8, DH), jnp.float32),
        ],
    )(xr, xat, tW, tb)
    return idx, h.reshape(N, 2 * DH), st


TM = 2048


def _norm_body(relu, pad, h_ref, st_ref, g_ref, b_ref, o_ref):
    m = st_ref[0:1, :] / N
    v = st_ref[1:2, :] / N - m * m
    o = (h_ref[...] - m) / jnp.sqrt(v + EPS) * g_ref[...] + b_ref[...]
    if relu:
        o = jnp.maximum(o, 0.0)
    if pad:
        o_ref[:, 0:DH] = o
    else:
        o_ref[...] = o


def _normalize(h, st, g, b, relu, pad=False):
    w = 2 * DH if pad else DH
    return pl.pallas_call(
        functools.partial(_norm_body, relu, pad),
        grid=(N // TM,),
        in_specs=[
            pl.BlockSpec((TM, DH), lambda i: (i, 0)),
            pl.BlockSpec((8, DH), lambda i: (0, 0)),
            pl.BlockSpec((1, DH), lambda i: (0, 0)),
            pl.BlockSpec((1, DH), lambda i: (0, 0)),
        ],
        out_specs=pl.BlockSpec((TM, w), lambda i: (i, 0)),
        out_shape=jax.ShapeDtypeStruct((N, w), jnp.float32),
    )(h, st, g, b)


CN = 32
IR = CN * K // 128
NPW = N // 32
NCHUNK = NPW // CN


NSLOT = 5
WROWS = NPW * K // 128


def _agg_sc_body(table_hbm, idx_hbm, out_hbm, idx_v, rows_v, acc_v, *sems):
    nc = 2
    wid = lax.axis_index("s") * nc + lax.axis_index("c")
    pltpu.sync_copy(idx_hbm.at[pl.ds(wid * WROWS, WROWS)], idx_v)

    def fire(s, sl):
        pltpu.async_copy(table_hbm.at[idx_v.at[s]],
                         rows_v.at[pl.ds(sl * 128, 128)], sems[sl])

    def wait(s, sl):
        pltpu.make_async_copy(table_hbm.at[idx_v.at[s]],
                              rows_v.at[pl.ds(sl * 128, 128)],
                              sems[sl]).wait()

    for sl in range(NSLOT - 1):
        fire(sl, sl)

    def group(g, carry):
        for sl in range(NSLOT):
            s = g * NSLOT + sl
            wait(s, sl)
            lo = (128 * s) // K
            hi = (128 * (s + 1)) // K

            def node(n, c):
                rb = n * K - g * (NSLOT * 128)
                for q in range(DH // 16):
                    f = pl.ds(16 * q, 16)
                    acc = rows_v[rb, f]
                    for k in range(1, K):
                        acc = jnp.maximum(acc, rows_v[rb + k, f])
                    acc_v[n - g * CN, f] = acc
                return c

            lax.fori_loop(lo, hi, node, None)

            @pl.when(s + NSLOT - 1 < WROWS)
            def _():
                fire(s + NSLOT - 1, (sl + NSLOT - 1) % NSLOT)

        node0 = wid * NPW + g * CN
        pltpu.sync_copy(acc_v, out_hbm.at[pl.ds(node0, CN)])
        return carry

    lax.fori_loop(0, NCHUNK, group, None)


def _agg_max(table, idx2d):
    mesh = plsc.VectorSubcoreMesh(core_axis_name="c", subcore_axis_name="s")
    kfn = pl.kernel(
        _agg_sc_body,
        mesh=mesh,
        out_type=jax.ShapeDtypeStruct((N, DH), jnp.float32),
        scratch_types=[
            pltpu.VMEM((WROWS, 128), jnp.int32),
            pltpu.VMEM((NSLOT * 128, 2 * DH), jnp.float32),
            pltpu.VMEM((CN, DH), jnp.float32),
        ] + [pltpu.SemaphoreType.DMA] * NSLOT,
    )
    return kfn(table, idx2d)


def _gin_body(relu_bn, pad, agg_ref, hself_ref, st_ref, g_ref, b_ref,
              wa_ref, ba_ref, wb_ref, bb_ref, y_ref, stout_ref):
    i = pl.program_id(0)
    a = jnp.maximum(agg_ref[...], hself_ref[:, 0:DH])
    m = st_ref[0:1, :] / N
    v = st_ref[1:2, :] / N - m * m
    a = (a - m) / jnp.sqrt(v + EPS) * g_ref[...] + b_ref[...]
    if relu_bn:
        a = jnp.maximum(a, 0.0)
    z = jnp.dot(a, wa_ref[...], preferred_element_type=jnp.float32)
    z = jnp.maximum(z + ba_ref[...], 0.0)
    y = jnp.dot(z, wb_ref[...], preferred_element_type=jnp.float32)
    y = y + bb_ref[...]
    if pad:
        y_ref[:, 0:DH] = y
    else:
        y_ref[...] = y

    @pl.when(i == 0)
    def _():
        stout_ref[...] = jnp.zeros_like(stout_ref)

    stout_ref[0:1, :] += jnp.sum(y, axis=0, keepdims=True)
    stout_ref[1:2, :] += jnp.sum(y * y, axis=0, keepdims=True)


def _gin(agg, hself, st, g, b, wa, ba, wb, bb, relu_bn, pad):
    w = 2 * DH if pad else DH
    return pl.pallas_call(
        functools.partial(_gin_body, relu_bn, pad),
        grid=(N // TM,),
        in_specs=[
            pl.BlockSpec((TM, DH), lambda i: (i, 0)),
            pl.BlockSpec((TM, 2 * DH), lambda i: (i, 0)),
            pl.BlockSpec((8, DH), lambda i: (0, 0)),
            pl.BlockSpec((1, DH), lambda i: (0, 0)),
            pl.BlockSpec((1, DH), lambda i: (0, 0)),
            pl.BlockSpec((DH, 2 * DH), lambda i: (0, 0)),
            pl.BlockSpec((1, 2 * DH), lambda i: (0, 0)),
            pl.BlockSpec((2 * DH, DH), lambda i: (0, 0)),
            pl.BlockSpec((1, DH), lambda i: (0, 0)),
        ],
        out_specs=[
            pl.BlockSpec((TM, w), lambda i: (i, 0)),
            pl.BlockSpec((8, DH), lambda i: (0, 0)),
        ],
        out_shape=[
            jax.ShapeDtypeStruct((N, w), jnp.float32),
            jax.ShapeDtypeStruct((8, DH), jnp.float32),
        ],
    )(agg, hself, st, g, b, wa, ba, wb, bb)


def kernel(x, batch, tW, tb, tg, tbeta, w1a, b1a, w1b, b1b, bn1g, bn1b,
           w2a, b2a, w2b, b2b, bn2g, bn2b):
    del batch
    r1 = lambda a: a.reshape(1, -1)

    idx, h0raw, st0 = _knn(x, tW, r1(tb))
    idx2d = idx.reshape(N * K // 128, 128)
    agg1 = _agg_max(h0raw, idx2d)
    y1raw, st1 = _gin(agg1, h0raw, st0, r1(tg), r1(tbeta),
                      w1a, r1(b1a), w1b, r1(b1b), relu_bn=False, pad=True)
    agg2 = _agg_max(y1raw, idx2d)
    y2raw, st2 = _gin(agg2, y1raw, st1, r1(bn1g), r1(bn1b),
                      w2a, r1(b2a), w2b, r1(b2b), relu_bn=True, pad=False)
    out = _normalize(y2raw, st2, r1(bn2g), r1(bn2b), relu=False)
    return out


# speedup vs baseline: 1.0184x; 1.0184x over previous
import functools

import jax
import jax.numpy as jnp
from jax import lax
from jax.experimental import pallas as pl
from jax.experimental.pallas import tpu as pltpu
from jax.experimental.pallas import tpu_sc as plsc

N = 32768
P = 2048
B = 16
K = 20
DIN = 3
DH = 64

R = 512
EPS = 1e-5


def _knn_body(xq_ref, xat_ref, tW_ref, tb_ref, idx_ref, h_ref, st_ref):
    b = pl.program_id(0)
    xq = xq_ref[0]
    xat = xat_ref[0]

    h = jnp.dot(xq, tW_ref[...], preferred_element_type=jnp.float32)
    h = jnp.maximum(h + tb_ref[...], 0.0)
    h_ref[0, :, 0:DH] = h

    @pl.when(jnp.logical_and(b == 0, pl.program_id(1) == 0))
    def _():
        st_ref[...] = jnp.zeros_like(st_ref)

    st_ref[0:1, :] += jnp.sum(h, axis=0, keepdims=True)
    st_ref[1:2, :] += jnp.sum(h * h, axis=0, keepdims=True)
    d2 = ((xq[:, 0:1] - xat[0:1, :]) ** 2
          + (xq[:, 1:2] - xat[1:2, :]) ** 2
          + (xq[:, 2:3] - xat[2:3, :]) ** 2)
    colio = lax.broadcasted_iota(jnp.int32, (R, P), 1)
    colf = colio.astype(jnp.float32)
    r = pl.program_id(1)
    selfcol = r * R + lax.broadcasted_iota(jnp.int32, (R, 1), 0)
    d2 = jnp.where(colio == selfcol, jnp.inf, d2)
    cols = [selfcol.astype(jnp.float32)]
    for k in range(K - 1):
        m = jnp.min(d2, axis=1, keepdims=True)
        eq = d2 == m
        t = jnp.where(eq, colf, float(P))
        a = jnp.min(t, axis=1, keepdims=True)
        cols.append(a)
        if k < K - 2:
            d2 = jnp.where(eq, jnp.inf, d2)
    idx = jnp.concatenate(cols, axis=1).astype(jnp.int32)
    idx_ref[0] = idx + b * P


def _knn(x, tW, tb):
    xr = x.reshape(B, P, DIN)
    xat = xr.transpose(0, 2, 1)
    idx, h, st = pl.pallas_call(
        _knn_body,
        grid=(B, P // R),
        in_specs=[
            pl.BlockSpec((1, R, DIN), lambda b, r: (b, r, 0)),
            pl.BlockSpec((1, DIN, P), lambda b, r: (b, 0, 0)),
            pl.BlockSpec((DIN, DH), lambda b, r: (0, 0)),
            pl.BlockSpec((1, DH), lambda b, r: (0, 0)),
        ],
        out_specs=[
            pl.BlockSpec((1, R, K), lambda b, r: (b, r, 0)),
            pl.BlockSpec((1, R, 2 * DH), lambda b, r: (b, r, 0)),
            pl.BlockSpec((8, DH), lambda b, r: (0, 0)),
        ],
        out_shape=[
            jax.ShapeDtypeStruct((B, P, K), jnp.int32),
            jax.ShapeDtypeStruct((B, P, 2 * DH), jnp.float32),
            jax.ShapeDtypeStruct((8, DH), jnp.float32),
        ],
    )(xr, xat, tW, tb)
    return idx, h.reshape(N, 2 * DH), st


TM = 2048


def _norm_body(relu, pad, h_ref, st_ref, g_ref, b_ref, o_ref):
    m = st_ref[0:1, :] / N
    v = st_ref[1:2, :] / N - m * m
    o = (h_ref[...] - m) / jnp.sqrt(v + EPS) * g_ref[...] + b_ref[...]
    if relu:
        o = jnp.maximum(o, 0.0)
    if pad:
        o_ref[:, 0:DH] = o
    else:
        o_ref[...] = o


def _normalize(h, st, g, b, relu, pad=False):
    w = 2 * DH if pad else DH
    return pl.pallas_call(
        functools.partial(_norm_body, relu, pad),
        grid=(N // TM,),
        in_specs=[
            pl.BlockSpec((TM, DH), lambda i: (i, 0)),
            pl.BlockSpec((8, DH), lambda i: (0, 0)),
            pl.BlockSpec((1, DH), lambda i: (0, 0)),
            pl.BlockSpec((1, DH), lambda i: (0, 0)),
        ],
        out_specs=pl.BlockSpec((TM, w), lambda i: (i, 0)),
        out_shape=jax.ShapeDtypeStruct((N, w), jnp.float32),
    )(h, st, g, b)


CN = 32
IR = CN * K // 128
NPW = N // 32
NCHUNK = NPW // CN


NSLOT = 5
WROWS = NPW * K // 128


def _agg_sc_body(table_hbm, idx_hbm, out_hbm, idx_v, rows_v, acc_v, *sems):
    nc = 2
    wid = lax.axis_index("s") * nc + lax.axis_index("c")
    pltpu.sync_copy(idx_hbm.at[pl.ds(wid * WROWS, WROWS)], idx_v)

    def fire(s, sl):
        pltpu.async_copy(table_hbm.at[idx_v.at[s]],
                         rows_v.at[pl.ds(sl * 128, 128)], sems[sl])

    def wait(s, sl):
        pltpu.make_async_copy(table_hbm.at[idx_v.at[s]],
                              rows_v.at[pl.ds(sl * 128, 128)],
                              sems[sl]).wait()

    for sl in range(NSLOT - 1):
        fire(sl, sl)

    def group(g, carry):
        for sl in range(NSLOT):
            s = g * NSLOT + sl
            wait(s, sl)
            lo = (128 * s) // K
            hi = (128 * (s + 1)) // K

            def node(n, c):
                rb = n * K - g * (NSLOT * 128)
                for q in range(DH // 16):
                    f = pl.ds(16 * q, 16)
                    acc = rows_v[rb, f]
                    for k in range(1, K):
                        acc = jnp.maximum(acc, rows_v[rb + k, f])
                    acc_v[n - g * CN, f] = acc
                return c

            lax.fori_loop(lo, hi, node, None)

            @pl.when(s + NSLOT - 1 < WROWS)
            def _():
                fire(s + NSLOT - 1, (sl + NSLOT - 1) % NSLOT)

        node0 = wid * NPW + g * CN
        pltpu.sync_copy(acc_v, out_hbm.at[pl.ds(node0, CN)])
        return carry

    lax.fori_loop(0, NCHUNK, group, None)


def _agg_max(table, idx2d):
    mesh = plsc.VectorSubcoreMesh(core_axis_name="c", subcore_axis_name="s")
    kfn = pl.kernel(
        _agg_sc_body,
        mesh=mesh,
        out_type=jax.ShapeDtypeStruct((N, DH), jnp.float32),
        scratch_types=[
            pltpu.VMEM((WROWS, 128), jnp.int32),
            pltpu.VMEM((NSLOT * 128, 2 * DH), jnp.float32),
            pltpu.VMEM((CN, DH), jnp.float32),
        ] + [pltpu.SemaphoreType.DMA] * NSLOT,
    )
    return kfn(table, idx2d)


def _gin_body(relu_bn, pad, agg_ref, hself_ref, st_ref, g_ref, b_ref,
              wa_ref, ba_ref, wb_ref, bb_ref, y_ref, stout_ref):
    i = pl.program_id(0)
    a = jnp.maximum(agg_ref[...], hself_ref[:, 0:DH])
    m = st_ref[0:1, :] / N
    v = st_ref[1:2, :] / N - m * m
    a = (a - m) / jnp.sqrt(v + EPS) * g_ref[...] + b_ref[...]
    if relu_bn:
        a = jnp.maximum(a, 0.0)
    z = jnp.dot(a, wa_ref[...], preferred_element_type=jnp.float32)
    z = jnp.maximum(z + ba_ref[...], 0.0)
    y = jnp.dot(z, wb_ref[...], preferred_element_type=jnp.float32)
    y = y + bb_ref[...]
    if pad:
        y_ref[:, 0:DH] = y
    else:
        y_ref[...] = y

    @pl.when(i == 0)
    def _():
        stout_ref[...] = jnp.zeros_like(stout_ref)

    stout_ref[0:1, :] += jnp.sum(y, axis=0, keepdims=True)
    stout_ref[1:2, :] += jnp.sum(y * y, axis=0, keepdims=True)


def _gin(agg, hself, st, g, b, wa, ba, wb, bb, relu_bn, pad):
    w = 2 * DH if pad else DH
    return pl.pallas_call(
        functools.partial(_gin_body, relu_bn, pad),
        grid=(N // TM,),
        in_specs=[
            pl.BlockSpec((TM, DH), lambda i: (i, 0)),
            pl.BlockSpec((TM, 2 * DH), lambda i: (i, 0)),
            pl.BlockSpec((8, DH), lambda i: (0, 0)),
            pl.BlockSpec((1, DH), lambda i: (0, 0)),
            pl.BlockSpec((1, DH), lambda i: (0, 0)),
            pl.BlockSpec((DH, 2 * DH), lambda i: (0, 0)),
            pl.BlockSpec((1, 2 * DH), lambda i: (0, 0)),
            pl.BlockSpec((2 * DH, DH), lambda i: (0, 0)),
            pl.BlockSpec((1, DH), lambda i: (0, 0)),
        ],
        out_specs=[
            pl.BlockSpec((TM, w), lambda i: (i, 0)),
            pl.BlockSpec((8, DH), lambda i: (0, 0)),
        ],
        out_shape=[
            jax.ShapeDtypeStruct((N, w), jnp.float32),
            jax.ShapeDtypeStruct((8, DH), jnp.float32),
        ],
    )(agg, hself, st, g, b, wa, ba, wb, bb)


def kernel(x, batch, tW, tb, tg, tbeta, w1a, b1a, w1b, b1b, bn1g, bn1b,
           w2a, b2a, w2b, b2b, bn2g, bn2b):
    del batch
    r1 = lambda a: a.reshape(1, -1)

    idx, h0raw, st0 = _knn(x, tW, r1(tb))
    idx2d = idx.reshape(N * K // 128, 128)
    agg1 = _agg_max(h0raw, idx2d)
    y1raw, st1 = _gin(agg1, h0raw, st0, r1(tg), r1(tbeta),
                      w1a, r1(b1a), w1b, r1(b1b), relu_bn=False, pad=True)
    agg2 = _agg_max(y1raw, idx2d)
    y2raw, st2 = _gin(agg2, y1raw, st1, r1(bn1g), r1(bn1b),
                      w2a, r1(b2a), w2b, r1(b2b), relu_bn=True, pad=False)
    out = _normalize(y2raw, st2, r1(bn2g), r1(bn2b), relu=False)
    return out
